# baseline (speedup 1.0000x reference)
import jax
import jax.numpy as jnp
from jax.experimental import pallas as pl
from jax.experimental.pallas import tpu as pltpu

_N_PRE = 2000
_PAD = 2048
_N_POST = 1000
_NMS_T = 0.7
_MIN = 1e-3
_IMG = 1024.0
_BIG = 1e9


def _rpn_nms_body(bxt_ref, bxn_ref, sc_ref, out_ref, cl_ref, alive_ref):
    x1 = jnp.clip(bxt_ref[0:1, :], 0.0, _IMG)
    y1 = jnp.clip(bxt_ref[1:2, :], 0.0, _IMG)
    x2 = jnp.clip(bxt_ref[2:3, :], 0.0, _IMG)
    y2 = jnp.clip(bxt_ref[3:4, :], 0.0, _IMG)
    probs = sc_ref[0:1, :]
    ws = x2 - x1
    hs = y2 - y1
    valid = (ws >= _MIN) & (hs >= _MIN) & (probs > 0.0)
    area = (x2 - x1) * (y2 - y1)
    cl_ref[0:1, :] = x1
    cl_ref[1:2, :] = y1
    cl_ref[2:3, :] = x2
    cl_ref[3:4, :] = y2
    cl_ref[4:5, :] = area
    alive_ref[...] = jnp.where(valid, probs, -_BIG)
    out_ref[...] = jnp.zeros(out_ref.shape, jnp.float32)

    cols8 = jax.lax.broadcasted_iota(jnp.int32, (1, 8), 1)
    lane = jax.lax.broadcasted_iota(jnp.int32, (1, _PAD), 1)

    def first_argmax():
        av = alive_ref[...]
        m = jnp.max(av)
        idx = jnp.min(jnp.where(av == m, lane, _PAD))
        return m, idx

    def cond(state):
        c, m, idx = state
        return (c < _N_POST) & (m > (-_BIG / 2.0))

    def step(state):
        c, m, idx = state
        xs1 = cl_ref[0:1, :]
        ys1 = cl_ref[1:2, :]
        xs2 = cl_ref[2:3, :]
        ys2 = cl_ref[3:4, :]
        ar = cl_ref[4:5, :]
        rowv = jnp.clip(bxn_ref[pl.ds(idx, 1), :], 0.0, _IMG)
        xi1 = rowv[0, 0]
        yi1 = rowv[0, 1]
        xi2 = rowv[0, 2]
        yi2 = rowv[0, 3]
        ai = (xi2 - xi1) * (yi2 - yi1)
        xx1 = jnp.maximum(xi1, xs1)
        yy1 = jnp.maximum(yi1, ys1)
        xx2 = jnp.minimum(xi2, xs2)
        yy2 = jnp.minimum(yi2, ys2)
        inter = jnp.clip(xx2 - xx1, 0.0, None) * jnp.clip(yy2 - yy1, 0.0, None)
        iou = inter / (ai + ar - inter + 1e-9)
        supp = iou > _NMS_T
        alive_ref[...] = jnp.where(supp, -_BIG, alive_ref[...])
        row = (jnp.where(cols8 == 0, xi1, 0.0)
               + jnp.where(cols8 == 1, yi1, 0.0)
               + jnp.where(cols8 == 2, xi2, 0.0)
               + jnp.where(cols8 == 3, yi2, 0.0)
               + jnp.where(cols8 == 4, m, 0.0))
        out_ref[pl.ds(c, 1), :] = row
        nm, nidx = first_argmax()
        return (c + 1, nm, nidx)

    m0, idx0 = first_argmax()
    jax.lax.while_loop(cond, step, (0, m0, idx0))


def kernel(boxes, scores):
    probs = jax.nn.sigmoid(scores)
    top_probs, idx = jax.lax.top_k(probs, _N_PRE)
    top_boxes = jnp.take(boxes, idx, axis=0)

    bxt = jnp.zeros((8, _PAD), jnp.float32).at[0:4, :_N_PRE].set(top_boxes.T)
    bxn = jnp.zeros((_PAD, 8), jnp.float32).at[:_N_PRE, 0:4].set(top_boxes)
    sc = jnp.full((1, _PAD), -1.0, jnp.float32).at[0, :_N_PRE].set(top_probs)

    out = pl.pallas_call(
        _rpn_nms_body,
        out_shape=jax.ShapeDtypeStruct((1024, 8), jnp.float32),
        scratch_shapes=[
            pltpu.VMEM((8, _PAD), jnp.float32),
            pltpu.VMEM((1, _PAD), jnp.float32),
        ],
    )(bxt, bxn, sc)
    return out[:_N_POST, :5]
